# baseline (speedup 1.0000x reference)
import functools

import jax
import jax.numpy as jnp
from jax import lax
from jax.experimental import pallas as pl
from jax.experimental.pallas import tpu as pltpu
from jax.experimental.pallas import tpu_sc as plsc

NINP = 64
LANES = 128
SEG = 128
NBUF = 3


def _make_emb(batch, hist, nw):
    bpw = batch // nw
    total = batch * hist
    mesh = plsc.VectorSubcoreMesh(core_axis_name="c", subcore_axis_name="s")

    @functools.partial(
        pl.kernel,
        mesh=mesh,
        out_type=jax.ShapeDtypeStruct((total, LANES), jnp.float32),
        scratch_types=[
            pltpu.VMEM((bpw * hist,), jnp.int32),
            pltpu.VMEM((NBUF, hist, LANES), jnp.float32),
            pltpu.SemaphoreType.DMA,
        ]
        + [pltpu.SemaphoreType.DMA] * NBUF
        + [pltpu.SemaphoreType.DMA] * NBUF,
        compiler_params=pltpu.CompilerParams(use_tc_tiling_on_sc=True),
    )
    def emb(idx_hbm, table_hbm, out_hbm, idx_v, rows_v, sem_i, *sems):
        sem_g = sems[:NBUF]
        sem_w = sems[NBUF:]
        wid = lax.axis_index("s") * 2 + lax.axis_index("c")
        base_b = wid * bpw

        def fire(s, b):
            pltpu.async_copy(
                table_hbm.at[idx_v.at[pl.ds(s * hist, SEG)]],
                rows_v.at[b, pl.ds(0, SEG)],
                sem_g[b],
            )
            pltpu.async_copy(
                table_hbm.at[idx_v.at[pl.ds(s * hist + SEG, hist - SEG)]],
                rows_v.at[b, pl.ds(SEG, hist - SEG)],
                sem_g[b],
            )

        def drain(b):
            pltpu.make_async_copy(
                table_hbm.at[idx_v.at[pl.ds(0, SEG)]],
                rows_v.at[b, pl.ds(0, SEG)],
                sem_g[b],
            ).wait()
            pltpu.make_async_copy(
                table_hbm.at[idx_v.at[pl.ds(0, hist - SEG)]],
                rows_v.at[b, pl.ds(SEG, hist - SEG)],
                sem_g[b],
            ).wait()

        def issue_write(s, b):
            pltpu.async_copy(
                rows_v.at[b],
                out_hbm.at[pl.ds((base_b + s) * hist, hist)],
                sem_w[b],
            )

        def wait_write(b):
            pltpu.make_async_copy(
                rows_v.at[b],
                out_hbm.at[pl.ds(0, hist)],
                sem_w[b],
            ).wait()

        pltpu.async_copy(idx_hbm.at[pl.ds(base_b * hist, bpw * hist)], idx_v, sem_i).wait()

        fire(0, 0)

        def body(s, carry):
            for bn in range(NBUF):

                @pl.when(((s + 1) % NBUF == bn) & (s + 1 < bpw))
                def _():
                    @pl.when(s + 1 >= NBUF)
                    def _():
                        wait_write(bn)

                    fire(s + 1, bn)

            for b in range(NBUF):

                @pl.when(s % NBUF == b)
                def _():
                    drain(b)
                    issue_write(s, b)

            return carry

        lax.fori_loop(0, bpw, body, 0)
        for b in range(NBUF):
            wait_write(b)

    return emb


def kernel(input, weight):
    batch, hist = input.shape
    nw = 32
    idx = input.reshape(-1).astype(jnp.int32)
    table128 = jnp.concatenate([weight, weight], axis=1)
    out = _make_emb(batch, hist, nw)(idx, table128)
    return out[:, :NINP].reshape(batch, hist, NINP)


# speedup vs baseline: 1.1449x; 1.1449x over previous
import functools

import jax
import jax.numpy as jnp
from jax import lax
from jax.experimental import pallas as pl
from jax.experimental.pallas import tpu as pltpu
from jax.experimental.pallas import tpu_sc as plsc

NINP = 64
LANES = 128
SEG = 128
NBUF = 4


def _make_emb(batch, hist, nw):
    bpw = batch // nw
    total = batch * hist
    mesh = plsc.VectorSubcoreMesh(core_axis_name="c", subcore_axis_name="s")

    @functools.partial(
        pl.kernel,
        mesh=mesh,
        out_type=jax.ShapeDtypeStruct((total, LANES), jnp.float32),
        scratch_types=[
            pltpu.VMEM((bpw * hist,), jnp.int32),
            pltpu.VMEM((NBUF, hist, LANES), jnp.float32),
            pltpu.SemaphoreType.DMA,
        ]
        + [pltpu.SemaphoreType.DMA] * NBUF
        + [pltpu.SemaphoreType.DMA] * NBUF,
        compiler_params=pltpu.CompilerParams(use_tc_tiling_on_sc=True),
    )
    def emb(idx_hbm, table_hbm, out_hbm, idx_v, rows_v, sem_i, *sems):
        sem_g = sems[:NBUF]
        sem_w = sems[NBUF:]
        wid = lax.axis_index("s") * 2 + lax.axis_index("c")
        base_b = wid * bpw

        def fire(s, b):
            pltpu.async_copy(
                table_hbm.at[idx_v.at[pl.ds(s * hist, SEG)]],
                rows_v.at[b, pl.ds(0, SEG)],
                sem_g[b],
            )
            pltpu.async_copy(
                table_hbm.at[idx_v.at[pl.ds(s * hist + SEG, hist - SEG)]],
                rows_v.at[b, pl.ds(SEG, hist - SEG)],
                sem_g[b],
            )

        def drain(b):
            pltpu.make_async_copy(
                table_hbm.at[idx_v.at[pl.ds(0, SEG)]],
                rows_v.at[b, pl.ds(0, SEG)],
                sem_g[b],
            ).wait()
            pltpu.make_async_copy(
                table_hbm.at[idx_v.at[pl.ds(0, hist - SEG)]],
                rows_v.at[b, pl.ds(SEG, hist - SEG)],
                sem_g[b],
            ).wait()

        def issue_write(s, b):
            pltpu.async_copy(
                rows_v.at[b],
                out_hbm.at[pl.ds((base_b + s) * hist, hist)],
                sem_w[b],
            )

        def wait_write(b):
            pltpu.make_async_copy(
                rows_v.at[b],
                out_hbm.at[pl.ds(0, hist)],
                sem_w[b],
            ).wait()

        pltpu.async_copy(idx_hbm.at[pl.ds(base_b * hist, bpw * hist)], idx_v, sem_i).wait()

        fire(0, 0)

        def body(s, carry):
            for bn in range(NBUF):

                @pl.when(((s + 1) % NBUF == bn) & (s + 1 < bpw))
                def _():
                    @pl.when(s + 1 >= NBUF)
                    def _():
                        wait_write(bn)

                    fire(s + 1, bn)

            for b in range(NBUF):

                @pl.when(s % NBUF == b)
                def _():
                    drain(b)
                    issue_write(s, b)

            return carry

        lax.fori_loop(0, bpw, body, 0)
        for b in range(NBUF):
            wait_write(b)

    return emb


def kernel(input, weight):
    batch, hist = input.shape
    nw = 32
    idx = input.reshape(-1).astype(jnp.int32)
    table128 = jnp.pad(weight, ((0, 0), (0, LANES - NINP)))
    out = _make_emb(batch, hist, nw)(idx, table128)
    return out[:, :NINP].reshape(batch, hist, NINP)


# speedup vs baseline: 1.1453x; 1.0003x over previous
import functools

import jax
import jax.numpy as jnp
from jax import lax
from jax.experimental import pallas as pl
from jax.experimental.pallas import tpu as pltpu
from jax.experimental.pallas import tpu_sc as plsc

NINP = 64
LANES = 128
SEG = 128
NBUF = 3


def _make_emb(batch, hist, nw):
    bpw = batch // nw
    total = batch * hist
    mesh = plsc.VectorSubcoreMesh(core_axis_name="c", subcore_axis_name="s")

    @functools.partial(
        pl.kernel,
        mesh=mesh,
        out_type=jax.ShapeDtypeStruct((total, LANES), jnp.float32),
        scratch_types=[
            pltpu.VMEM((bpw * hist,), jnp.int32),
            pltpu.VMEM((NBUF, hist, LANES), jnp.float32),
            pltpu.SemaphoreType.DMA,
        ]
        + [pltpu.SemaphoreType.DMA] * NBUF
        + [pltpu.SemaphoreType.DMA] * NBUF,
        compiler_params=pltpu.CompilerParams(use_tc_tiling_on_sc=True),
    )
    def emb(idx_hbm, table_hbm, out_hbm, idx_v, rows_v, sem_i, *sems):
        sem_g = sems[:NBUF]
        sem_w = sems[NBUF:]
        wid = lax.axis_index("s") * 2 + lax.axis_index("c")
        base_b = wid * bpw

        def fire(s, b):
            pltpu.async_copy(
                table_hbm.at[idx_v.at[pl.ds(s * hist, SEG)]],
                rows_v.at[b, pl.ds(0, SEG)],
                sem_g[b],
            )
            pltpu.async_copy(
                table_hbm.at[idx_v.at[pl.ds(s * hist + SEG, hist - SEG)]],
                rows_v.at[b, pl.ds(SEG, hist - SEG)],
                sem_g[b],
            )

        def drain(b):
            pltpu.make_async_copy(
                table_hbm.at[idx_v.at[pl.ds(0, SEG)]],
                rows_v.at[b, pl.ds(0, SEG)],
                sem_g[b],
            ).wait()
            pltpu.make_async_copy(
                table_hbm.at[idx_v.at[pl.ds(0, hist - SEG)]],
                rows_v.at[b, pl.ds(SEG, hist - SEG)],
                sem_g[b],
            ).wait()

        def issue_write(s, b):
            pltpu.async_copy(
                rows_v.at[b],
                out_hbm.at[pl.ds((base_b + s) * hist, hist)],
                sem_w[b],
            )

        def wait_write(b):
            pltpu.make_async_copy(
                rows_v.at[b],
                out_hbm.at[pl.ds(0, hist)],
                sem_w[b],
            ).wait()

        pltpu.async_copy(idx_hbm.at[pl.ds(base_b * hist, bpw * hist)], idx_v, sem_i).wait()

        fire(0, 0)

        def body(s, carry):
            for bn in range(NBUF):

                @pl.when(((s + 1) % NBUF == bn) & (s + 1 < bpw))
                def _():
                    @pl.when(s + 1 >= NBUF)
                    def _():
                        wait_write(bn)

                    fire(s + 1, bn)

            for b in range(NBUF):

                @pl.when(s % NBUF == b)
                def _():
                    drain(b)
                    issue_write(s, b)

            return carry

        lax.fori_loop(0, bpw, body, 0)
        for b in range(NBUF):
            wait_write(b)

    return emb


def kernel(input, weight):
    batch, hist = input.shape
    nw = 32
    idx = input.reshape(-1).astype(jnp.int32)
    table128 = jnp.pad(weight, ((0, 0), (0, LANES - NINP)))
    out = _make_emb(batch, hist, nw)(idx, table128)
    return out[:, :NINP].reshape(batch, hist, NINP)
